# baseline (speedup 1.0000x reference)
import jax
import jax.numpy as jnp
from jax.experimental import pallas as pl


def _match_kernel(q_ref, dbt_ref, out_ref):
    d = q_ref.shape[-1]
    sq = jnp.where(q_ref[...] > 0, 1.0, -1.0).astype(jnp.bfloat16)
    sdbt = jnp.where(dbt_ref[...] > 0, 1.0, -1.0).astype(jnp.bfloat16)
    acc = jax.lax.dot_general(
        sq, sdbt, (((1,), (0,)), ((), ())), preferred_element_type=jnp.float32
    )
    out_ref[...] = (acc >= (d - 1.0)).astype(jnp.float32)


def kernel(queries, db):
    q, d = queries.shape
    n = db.shape[0]
    dbt = jnp.swapaxes(db, 0, 1)
    nb = 8192
    return pl.pallas_call(
        _match_kernel,
        grid=(n // nb,),
        in_specs=[
            pl.BlockSpec((q, d), lambda i: (0, 0)),
            pl.BlockSpec((d, nb), lambda i: (0, i)),
        ],
        out_specs=pl.BlockSpec((q, nb), lambda i: (0, i)),
        out_shape=jax.ShapeDtypeStruct((q, n), jnp.float32),
    )(queries, dbt)


# speedup vs baseline: 1.0043x; 1.0043x over previous
import jax
import jax.numpy as jnp
from jax.experimental import pallas as pl


def _match_kernel(q_ref, dbt_ref, out_ref):
    d = q_ref.shape[-1]
    sq = jnp.where(q_ref[...] > 0, 1.0, -1.0).astype(jnp.bfloat16)
    sdbt = jnp.where(dbt_ref[...] > 0, 1.0, -1.0).astype(jnp.bfloat16)
    acc = jax.lax.dot_general(
        sq, sdbt, (((1,), (0,)), ((), ())), preferred_element_type=jnp.float32
    )
    out_ref[...] = (acc >= (d - 1.0)).astype(jnp.float32)


def kernel(queries, db):
    q, d = queries.shape
    n = db.shape[0]
    dbt = jnp.swapaxes(db, 0, 1)
    nb = 32768
    qb = 128
    return pl.pallas_call(
        _match_kernel,
        grid=(n // nb, q // qb),
        in_specs=[
            pl.BlockSpec((qb, d), lambda j, i: (i, 0)),
            pl.BlockSpec((d, nb), lambda j, i: (0, j)),
        ],
        out_specs=pl.BlockSpec((qb, nb), lambda j, i: (i, j)),
        out_shape=jax.ShapeDtypeStruct((q, n), jnp.float32),
    )(queries, dbt)


# speedup vs baseline: 1.0385x; 1.0341x over previous
import jax
import jax.numpy as jnp
from jax.experimental import pallas as pl


def _match_kernel(q_ref, dbt_ref, out_ref):
    d = q_ref.shape[-1]
    sq = jnp.where(q_ref[...] > 0, 1.0, -1.0).astype(jnp.bfloat16)
    sdbt = jnp.where(dbt_ref[...] > 0, 1.0, -1.0).astype(jnp.bfloat16)
    acc = jax.lax.dot_general(
        sq, sdbt, (((1,), (0,)), ((), ())), preferred_element_type=jnp.float32
    )
    out_ref[...] = (acc >= (d - 1.0)).astype(jnp.float32)


def kernel(queries, db):
    q, d = queries.shape
    n = db.shape[0]
    dbt = jnp.swapaxes(db, 0, 1)
    nb = 16384
    while n % nb:
        nb //= 2
    return pl.pallas_call(
        _match_kernel,
        grid=(n // nb,),
        in_specs=[
            pl.BlockSpec((q, d), lambda i: (0, 0)),
            pl.BlockSpec((d, nb), lambda i: (0, i)),
        ],
        out_specs=pl.BlockSpec((q, nb), lambda i: (0, i)),
        out_shape=jax.ShapeDtypeStruct((q, n), jnp.float32),
    )(queries, dbt)
